# baseline (speedup 1.0000x reference)
import dataclasses
import functools

import jax
import jax.numpy as jnp
from jax import lax
from jax.experimental import pallas as pl
from jax.experimental.pallas import tpu as pltpu
from jax.experimental.pallas import tpu_sc as plsc

NUM_USERS = 40000
NUM_ITEMS = 60000
D = 32
HD = 16
N_TOTAL = NUM_USERS + NUM_ITEMS
E_GRAPH = 1600000
E_SOCIAL = 640000

G_ROWS = 12672
G_INNER = 6
G_CHUNKS = G_ROWS // G_INNER
G_PAIRS = 66
ACC_ROWS = 100096

S_ROWS = 5056
S_REAL_ROWS = E_SOCIAL // 128
S_CHUNKS = S_ROWS // 2
S_PAIRS = 79
U_ACC_ROWS = 40064
U_GARBAGE = NUM_USERS

_MESH = plsc.VectorSubcoreMesh(core_axis_name="c", subcore_axis_name="s")

_SC_PARAMS = pltpu.CompilerParams()
if "needs_layout_passes" in pltpu.CompilerParams.__dataclass_fields__:
    _SC_PARAMS = dataclasses.replace(_SC_PARAMS, needs_layout_passes=False)
if "use_tc_tiling_on_sc" in pltpu.CompilerParams.__dataclass_fields__:
    _SC_PARAMS = dataclasses.replace(_SC_PARAMS, use_tc_tiling_on_sc=False)


def _spmm(table3, dst3, src3, val3):
    @functools.partial(
        pl.kernel,
        out_type=jax.ShapeDtypeStruct((2, N_TOTAL, HD), jnp.float32),
        mesh=_MESH,
        compiler_params=_SC_PARAMS,
        scratch_types=[
            pltpu.VMEM((2, G_INNER, 128), jnp.int32),
            pltpu.VMEM((2, G_INNER, 128), jnp.int32),
            pltpu.VMEM((2, G_INNER, 128), jnp.float32),
            pltpu.VMEM((2, G_INNER * 128, HD), jnp.float32),
            pltpu.VMEM_SHARED((ACC_ROWS, HD), jnp.float32),
            pltpu.SemaphoreType.DMA,
            pltpu.SemaphoreType.DMA,
            pltpu.SemaphoreType.DMA,
            pltpu.SemaphoreType.DMA,
        ],
    )
    def k(table_h, dst_h, src_h, val_h, out_h, dstv, srcv, valv,
          rows, acc, semi0, semi1, semg0, semg1):
        c = lax.axis_index("c")
        s = lax.axis_index("s")
        zero16 = jnp.zeros((16,), jnp.float32)
        semi = (semi0, semi1)
        semg = (semg0, semg1)

        def idx_cps(t, p):
            return [pltpu.make_async_copy(dst_h.at[t], dstv.at[p], semi[p]),
                    pltpu.make_async_copy(src_h.at[t], srcv.at[p], semi[p]),
                    pltpu.make_async_copy(val_h.at[t], valv.at[p], semi[p])]

        def fire(cps):
            for cp in cps:
                cp.start()

        def wait(cps):
            for cp in cps:
                cp.wait()

        def process(p):
            pv = jnp.full((16,), p, jnp.int32)
            for j in range(G_INNER):
                cj = jnp.full((16,), j, jnp.int32)

                @pl.loop(0, 32)
                def _(q, _j=j, _cj=cj, _p=p, _pv=pv):
                    i0 = q * 4
                    for u in range(4):
                        vv = plsc.load_gather(
                            valv,
                            [_pv, _cj, jnp.full((16,), i0 + u, jnp.int32)])
                        r = _j * 128 + i0 + u
                        rows[_p, r, pl.ds(0, 16)] = (
                            rows[_p, r, pl.ds(0, 16)] * vv)
            for j in range(G_INNER):
                pltpu.sync_copy(rows.at[p, pl.ds(j * 128, 128)],
                                acc.at[dstv.at[p, j]], add=True)

        if True:
            def gat_cps(p):
                return [pltpu.make_async_copy(
                            table_h.at[c].at[srcv.at[p, j]],
                            rows.at[p, pl.ds(j * 128, 128)], semg[p])
                        for j in range(G_INNER)]

            @pl.loop(0, 768)
            def _(i):
                rows[0, i, pl.ds(0, 16)] = zero16

            zb = s * 6256
            for kk in range(8):
                pltpu.sync_copy(rows.at[0, pl.ds(0, 768)],
                                acc.at[pl.ds(zb + kk * 768, 768)])
            pltpu.sync_copy(rows.at[0, pl.ds(0, 112)],
                            acc.at[pl.ds(zb + 6144, 112)])
            plsc.subcore_barrier()

            fire(idx_cps(s, 0))
            wait(idx_cps(s, 0))
            fire(gat_cps(0))
            fire(idx_cps(s + 16, 1))

            @pl.loop(0, G_PAIRS)
            def _pair(kk):
                a = s + kk * 32
                wait(idx_cps(a + 16, 1))
                fire(gat_cps(1))
                wait(gat_cps(0))
                process(0)

                @pl.when(kk < G_PAIRS - 1)
                def _():
                    fire(idx_cps(a + 32, 0))

                wait(gat_cps(1))

                @pl.when(kk < G_PAIRS - 1)
                def _():
                    wait(idx_cps(a + 32, 0))
                    fire(gat_cps(0))

                process(1)

                @pl.when(kk < G_PAIRS - 1)
                def _():
                    fire(idx_cps(a + 48, 1))

            plsc.subcore_barrier()
            wb = s * 6256

            @pl.when(s < 15)
            def _():
                pltpu.sync_copy(acc.at[pl.ds(wb, 6256)],
                                out_h.at[c, pl.ds(wb, 6256)])

            @pl.when(s == 15)
            def _():
                pltpu.sync_copy(acc.at[pl.ds(15 * 6256, 6160)],
                                out_h.at[c, pl.ds(15 * 6256, 6160)])

    return k(table3, dst3, src3, val3)


def _gat_edges(T3, F1, dst3, src3):
    @functools.partial(
        pl.kernel,
        out_type=jax.ShapeDtypeStruct((2, NUM_USERS, D), jnp.float32),
        mesh=_MESH,
        compiler_params=_SC_PARAMS,
        scratch_types=[
            pltpu.VMEM((2, 2, 128), jnp.int32),
            pltpu.VMEM((2, 2, 128), jnp.int32),
            pltpu.VMEM((2, 2, 128), jnp.int32),
            pltpu.VMEM((2, 256, D), jnp.float32),
            pltpu.VMEM((2, 256, 16), jnp.float32),
            pltpu.VMEM((2, 2, 128), jnp.float32),
            pltpu.VMEM_SHARED((U_ACC_ROWS, D), jnp.float32),
            pltpu.SemaphoreType.DMA,
            pltpu.SemaphoreType.DMA,
            pltpu.SemaphoreType.DMA,
            pltpu.SemaphoreType.DMA,
        ],
    )
    def k(T_h, F1_h, dst_h, src_h, G_h, dstv, srcv, sidx, S, Fb, ebuf,
          acc, semi0, semi1, semg0, semg1):
        c = lax.axis_index("c")
        s = lax.axis_index("s")
        zero16 = jnp.zeros((16,), jnp.float32)
        semi = (semi0, semi1)
        semg = (semg0, semg1)
        iota = lax.iota(jnp.int32, 16)
        col16 = jnp.full((16,), 16, jnp.int32)
        col0 = jnp.zeros((16,), jnp.int32)

        def idx_cps(t, p):
            return [pltpu.make_async_copy(dst_h.at[t], dstv.at[p], semi[p]),
                    pltpu.make_async_copy(src_h.at[t], srcv.at[p], semi[p])]

        def gat_cps(p):
            cps = []
            for j in range(2):
                cps.append(pltpu.make_async_copy(
                    T_h.at[c].at[srcv.at[p, j]],
                    S.at[p, pl.ds(j * 128, 128)], semg[p]))
                cps.append(pltpu.make_async_copy(
                    F1_h.at[dstv.at[p, j]],
                    Fb.at[p, pl.ds(j * 128, 128)], semg[p]))
            return cps

        def fire(cps):
            for cp in cps:
                cp.start()

        def wait(cps):
            for cp in cps:
                cp.wait()

        def process(t, p):
            pv = jnp.full((16,), p, jnp.int32)
            for j in range(2):
                real = jnp.full((16,), 1, jnp.int32) * (t * 2 + j) \
                    < S_REAL_ROWS
                for g in range(8):
                    dv = dstv[p, j, pl.ds(g * 16, 16)]
                    sidx[p, j, pl.ds(g * 16, 16)] = jnp.where(
                        real, dv, U_GARBAGE)
            for j in range(2):
                for g8 in range(8):
                    k0 = jnp.full((16,), j * 128 + g8 * 16, jnp.int32) + iota
                    f2v = plsc.load_gather(S, [pv, k0, col16])
                    f1v = plsc.load_gather(Fb, [pv, k0, col0])
                    sv = f1v + f2v
                    ev = jnp.exp(-jnp.maximum(sv, 0.2 * sv))
                    ebuf[p, j, pl.ds(g8 * 16, 16)] = ev
            for j in range(2):
                cj = jnp.full((16,), j, jnp.int32)

                @pl.loop(0, 32)
                def _(q, _j=j, _cj=cj, _p=p, _pv=pv):
                    i0 = q * 4
                    for u in range(4):
                        evv = plsc.load_gather(
                            ebuf,
                            [_pv, _cj, jnp.full((16,), i0 + u, jnp.int32)])
                        r = _j * 128 + i0 + u
                        S[_p, r, pl.ds(0, 16)] = S[_p, r, pl.ds(0, 16)] * evv
                        S[_p, r, pl.ds(16, 16)] = evv
            for j in range(2):
                pltpu.sync_copy(S.at[p, pl.ds(j * 128, 128)],
                                acc.at[sidx.at[p, j]], add=True)

        @pl.loop(0, 256)
        def _(i):
            S[0, i, pl.ds(0, 16)] = zero16
            S[0, i, pl.ds(16, 16)] = zero16

        zb = s * 2504
        for kk in range(9):
            pltpu.sync_copy(S.at[0, pl.ds(0, 256)],
                            acc.at[pl.ds(zb + kk * 256, 256)])
        pltpu.sync_copy(S.at[0, pl.ds(0, 200)], acc.at[pl.ds(zb + 2304, 200)])
        plsc.subcore_barrier()

        fire(idx_cps(s, 0))
        wait(idx_cps(s, 0))
        fire(gat_cps(0))
        fire(idx_cps(s + 16, 1))

        @pl.loop(0, S_PAIRS)
        def _pair(kk):
            a = s + kk * 32
            wait(idx_cps(a + 16, 1))
            fire(gat_cps(1))
            wait(gat_cps(0))
            process(a, 0)

            @pl.when(kk < S_PAIRS - 1)
            def _():
                fire(idx_cps(a + 32, 0))

            wait(gat_cps(1))

            @pl.when(kk < S_PAIRS - 1)
            def _():
                wait(idx_cps(a + 32, 0))
                fire(gat_cps(0))

            process(a + 16, 1)

            @pl.when(kk < S_PAIRS - 1)
            def _():
                fire(idx_cps(a + 48, 1))

        plsc.subcore_barrier()
        wb = s * 2504

        @pl.when(s < 15)
        def _():
            pltpu.sync_copy(acc.at[pl.ds(wb, 2504)],
                            G_h.at[c, pl.ds(wb, 2504)])

        @pl.when(s == 15)
        def _():
            pltpu.sync_copy(acc.at[pl.ds(15 * 2504, 2440)],
                            G_h.at[c, pl.ds(15 * 2504, 2440)])

    return k(T3, F1, dst3, src3)


def _mean_body(e0, e1, e2, e3, o):
    o[:, 0:HD] = (e0[0] + e1[0] + e2[0] + e3[0]) * 0.25
    o[:, HD:D] = (e0[1] + e1[1] + e2[1] + e3[1]) * 0.25


def _mean(e0, e1, e2, e3):
    blk = 4000
    bs = pl.BlockSpec((2, blk, HD), lambda i: (0, i, 0))
    return pl.pallas_call(
        _mean_body,
        grid=(N_TOTAL // blk,),
        in_specs=[bs] * 4,
        out_specs=pl.BlockSpec((blk, D), lambda i: (i, 0)),
        out_shape=jax.ShapeDtypeStruct((N_TOTAL, D), jnp.float32),
    )(e0, e1, e2, e3)


_UBLK = 2000
_UGRID = NUM_USERS // _UBLK


def _pre_body(x, W, ad, asrc, T, F1):
    h = lax.dot_general(x[...], W[...], (((1,), (0,)), ((), ())),
                        precision=lax.Precision.HIGHEST,
                        preferred_element_type=jnp.float32)
    f1 = jnp.sum(h * ad[...], axis=1, keepdims=True)
    f2 = jnp.sum(h * asrc[...], axis=1, keepdims=True)
    f2b = jnp.broadcast_to(f2, (_UBLK, 16))
    T[0, :, 0:16] = h[:, 0:16]
    T[0, :, 16:32] = f2b
    T[1, :, 0:16] = h[:, 16:32]
    T[1, :, 16:32] = f2b
    F1[...] = jnp.broadcast_to(f1, (_UBLK, 16))


def _pre(x, W, a):
    ad = a[:D, 0].reshape(1, D)
    asrc = a[D:, 0].reshape(1, D)
    full = pl.BlockSpec((1, D), lambda i: (0, 0))
    wfull = pl.BlockSpec((D, D), lambda i: (0, 0))
    return pl.pallas_call(
        _pre_body,
        grid=(_UGRID,),
        in_specs=[pl.BlockSpec((_UBLK, D), lambda i: (i, 0)),
                  wfull, full, full],
        out_specs=(pl.BlockSpec((2, _UBLK, D), lambda i: (0, i, 0)),
                   pl.BlockSpec((_UBLK, 16), lambda i: (i, 0))),
        out_shape=(jax.ShapeDtypeStruct((2, NUM_USERS, D), jnp.float32),
                   jax.ShapeDtypeStruct((NUM_USERS, 16), jnp.float32)),
    )(x, W, ad, asrc)


def _gat_norm(G):
    rs = G[0, :, 16:17] + 1e-9
    g = jnp.concatenate([G[0, :, 0:16], G[1, :, 0:16]], axis=1) / rs
    return jnp.where(g > 0, g, jnp.exp(g) - 1.0)


def _post_body(G, o):
    o[...] = _gat_norm(G)


def _post(G):
    return pl.pallas_call(
        _post_body,
        grid=(_UGRID,),
        in_specs=[pl.BlockSpec((2, _UBLK, D), lambda i: (0, i, 0))],
        out_specs=pl.BlockSpec((_UBLK, D), lambda i: (i, 0)),
        out_shape=jax.ShapeDtypeStruct((NUM_USERS, D), jnp.float32),
    )(G)


def _final_body(G, u, o):
    o[...] = (u[...] + _gat_norm(G)) * 0.5


def _final(G, u):
    return pl.pallas_call(
        _final_body,
        grid=(_UGRID,),
        in_specs=[pl.BlockSpec((2, _UBLK, D), lambda i: (0, i, 0)),
                  pl.BlockSpec((_UBLK, D), lambda i: (i, 0))],
        out_specs=pl.BlockSpec((_UBLK, D), lambda i: (i, 0)),
        out_shape=jax.ShapeDtypeStruct((NUM_USERS, D), jnp.float32),
    )(G, u)


def _pad3(x2d, rows, fill, inner):
    pad = rows - x2d.shape[0]
    full = jnp.concatenate(
        [x2d, jnp.full((pad, 128), fill, x2d.dtype)], axis=0)
    return full.reshape(rows // inner, inner, 128)


def kernel(graph_indices, graph_values, social_indices, user_emb, item_emb,
           W1, a1, W2, a2):
    dst3 = _pad3(graph_indices[0].reshape(-1, 128), G_ROWS, N_TOTAL, G_INNER)
    src3 = _pad3(graph_indices[1].reshape(-1, 128), G_ROWS, 0, G_INNER)
    val3 = _pad3(graph_values.reshape(-1, 128), G_ROWS, 0.0, G_INNER)
    sdst3 = _pad3(social_indices[0].reshape(-1, 128), S_ROWS, 0, 2)
    ssrc3 = _pad3(social_indices[1].reshape(-1, 128), S_ROWS, 0, 2)

    ego0 = jnp.stack([
        jnp.concatenate([user_emb[:, :HD], item_emb[:, :HD]], axis=0),
        jnp.concatenate([user_emb[:, HD:], item_emb[:, HD:]], axis=0)])
    e1 = _spmm(ego0, dst3, src3, val3)
    e2 = _spmm(e1, dst3, src3, val3)
    e3 = _spmm(e2, dst3, src3, val3)
    mean = _mean(ego0, e1, e2, e3)
    user_all = mean[:NUM_USERS]
    item_all = mean[NUM_USERS:]

    T1, F11 = _pre(user_all, W1, a1)
    G1 = _gat_edges(T1, F11, sdst3, ssrc3)
    h = _post(G1)
    T2, F12 = _pre(h, W2, a2)
    G2 = _gat_edges(T2, F12, sdst3, ssrc3)
    out_user = _final(G2, user_all)
    return (out_user, item_all)


# speedup vs baseline: 1.0371x; 1.0371x over previous
import dataclasses
import functools

import jax
import jax.numpy as jnp
from jax import lax
from jax.experimental import pallas as pl
from jax.experimental.pallas import tpu as pltpu
from jax.experimental.pallas import tpu_sc as plsc

NUM_USERS = 40000
NUM_ITEMS = 60000
D = 32
HD = 16
N_TOTAL = NUM_USERS + NUM_ITEMS
E_GRAPH = 1600000
E_SOCIAL = 640000

G_ROWS = 12544
G_CHUNKS = G_ROWS // 4
G_PAIRS = 98
ACC_ROWS = 100096

S_ROWS = 5056
S_REAL_ROWS = E_SOCIAL // 128
S_CHUNKS = S_ROWS // 2
S_PAIRS = 79
U_ACC_ROWS = 40064
U_GARBAGE = NUM_USERS

_MESH = plsc.VectorSubcoreMesh(core_axis_name="c", subcore_axis_name="s")

_SC_PARAMS = pltpu.CompilerParams()
if "needs_layout_passes" in pltpu.CompilerParams.__dataclass_fields__:
    _SC_PARAMS = dataclasses.replace(_SC_PARAMS, needs_layout_passes=False)
if "use_tc_tiling_on_sc" in pltpu.CompilerParams.__dataclass_fields__:
    _SC_PARAMS = dataclasses.replace(_SC_PARAMS, use_tc_tiling_on_sc=False)


def _spmm(table3, dst3, src3, val3):
    @functools.partial(
        pl.kernel,
        out_type=jax.ShapeDtypeStruct((2, N_TOTAL, HD), jnp.float32),
        mesh=_MESH,
        compiler_params=_SC_PARAMS,
        scratch_types=[
            pltpu.VMEM((2, 4, 128), jnp.int32),
            pltpu.VMEM((2, 4, 128), jnp.int32),
            pltpu.VMEM((2, 4, 128), jnp.float32),
            pltpu.VMEM((2, 512, HD), jnp.float32),
            pltpu.VMEM_SHARED((ACC_ROWS, HD), jnp.float32),
            pltpu.SemaphoreType.DMA,
            pltpu.SemaphoreType.DMA,
            pltpu.SemaphoreType.DMA,
            pltpu.SemaphoreType.DMA,
        ],
    )
    def k(table_h, dst_h, src_h, val_h, out_h, dstv, srcv, valv,
          rows, acc, semi0, semi1, semg0, semg1):
        c = lax.axis_index("c")
        s = lax.axis_index("s")
        zero16 = jnp.zeros((16,), jnp.float32)
        semi = (semi0, semi1)
        semg = (semg0, semg1)

        def idx_cps(t, p):
            return [pltpu.make_async_copy(dst_h.at[t], dstv.at[p], semi[p]),
                    pltpu.make_async_copy(src_h.at[t], srcv.at[p], semi[p]),
                    pltpu.make_async_copy(val_h.at[t], valv.at[p], semi[p])]

        def fire(cps):
            for cp in cps:
                cp.start()

        def wait(cps):
            for cp in cps:
                cp.wait()

        def process(p):
            pv = jnp.full((16,), p, jnp.int32)
            for j in range(4):
                cj = jnp.full((16,), j, jnp.int32)

                @pl.loop(0, 32)
                def _(q, _j=j, _cj=cj, _p=p, _pv=pv):
                    i0 = q * 4
                    for u in range(4):
                        vv = plsc.load_gather(
                            valv,
                            [_pv, _cj, jnp.full((16,), i0 + u, jnp.int32)])
                        r = _j * 128 + i0 + u
                        rows[_p, r, pl.ds(0, 16)] = (
                            rows[_p, r, pl.ds(0, 16)] * vv)
            for j in range(4):
                pltpu.sync_copy(rows.at[p, pl.ds(j * 128, 128)],
                                acc.at[dstv.at[p, j]], add=True)

        if True:
            def gat_cps(p):
                return [pltpu.make_async_copy(
                            table_h.at[c].at[srcv.at[p, j]],
                            rows.at[p, pl.ds(j * 128, 128)], semg[p])
                        for j in range(4)]

            @pl.loop(0, 512)
            def _(i):
                rows[0, i, pl.ds(0, 16)] = zero16

            zb = s * 6256
            for kk in range(12):
                pltpu.sync_copy(rows.at[0, pl.ds(0, 512)],
                                acc.at[pl.ds(zb + kk * 512, 512)])
            pltpu.sync_copy(rows.at[0, pl.ds(0, 112)],
                            acc.at[pl.ds(zb + 6144, 112)])
            plsc.subcore_barrier()

            fire(idx_cps(s, 0))
            wait(idx_cps(s, 0))
            fire(gat_cps(0))
            fire(idx_cps(s + 16, 1))

            @pl.loop(0, G_PAIRS)
            def _pair(kk):
                a = s + kk * 32
                wait(idx_cps(a + 16, 1))
                fire(gat_cps(1))
                wait(gat_cps(0))
                process(0)

                @pl.when(kk < G_PAIRS - 1)
                def _():
                    fire(idx_cps(a + 32, 0))

                wait(gat_cps(1))

                @pl.when(kk < G_PAIRS - 1)
                def _():
                    wait(idx_cps(a + 32, 0))
                    fire(gat_cps(0))

                process(1)

                @pl.when(kk < G_PAIRS - 1)
                def _():
                    fire(idx_cps(a + 48, 1))

            plsc.subcore_barrier()
            wb = s * 6256

            @pl.when(s < 15)
            def _():
                pltpu.sync_copy(acc.at[pl.ds(wb, 6256)],
                                out_h.at[c, pl.ds(wb, 6256)])

            @pl.when(s == 15)
            def _():
                pltpu.sync_copy(acc.at[pl.ds(15 * 6256, 6160)],
                                out_h.at[c, pl.ds(15 * 6256, 6160)])

    return k(table3, dst3, src3, val3)


def _gat_edges(T3, F1, dst3, src3):
    @functools.partial(
        pl.kernel,
        out_type=jax.ShapeDtypeStruct((2, NUM_USERS, D), jnp.float32),
        mesh=_MESH,
        compiler_params=_SC_PARAMS,
        scratch_types=[
            pltpu.VMEM((2, 2, 128), jnp.int32),
            pltpu.VMEM((2, 2, 128), jnp.int32),
            pltpu.VMEM((2, 2, 128), jnp.int32),
            pltpu.VMEM((2, 256, D), jnp.float32),
            pltpu.VMEM((2, 256, 16), jnp.float32),
            pltpu.VMEM((2, 2, 128), jnp.float32),
            pltpu.VMEM_SHARED((U_ACC_ROWS, D), jnp.float32),
            pltpu.SemaphoreType.DMA,
            pltpu.SemaphoreType.DMA,
            pltpu.SemaphoreType.DMA,
            pltpu.SemaphoreType.DMA,
        ],
    )
    def k(T_h, F1_h, dst_h, src_h, G_h, dstv, srcv, sidx, S, Fb, ebuf,
          acc, semi0, semi1, semg0, semg1):
        c = lax.axis_index("c")
        s = lax.axis_index("s")
        zero16 = jnp.zeros((16,), jnp.float32)
        semi = (semi0, semi1)
        semg = (semg0, semg1)
        iota = lax.iota(jnp.int32, 16)
        col16 = jnp.full((16,), 16, jnp.int32)
        col0 = jnp.zeros((16,), jnp.int32)

        def idx_cps(t, p):
            return [pltpu.make_async_copy(dst_h.at[t], dstv.at[p], semi[p]),
                    pltpu.make_async_copy(src_h.at[t], srcv.at[p], semi[p])]

        def gat_cps(p):
            cps = []
            for j in range(2):
                cps.append(pltpu.make_async_copy(
                    T_h.at[c].at[srcv.at[p, j]],
                    S.at[p, pl.ds(j * 128, 128)], semg[p]))
                cps.append(pltpu.make_async_copy(
                    F1_h.at[dstv.at[p, j]],
                    Fb.at[p, pl.ds(j * 128, 128)], semg[p]))
            return cps

        def fire(cps):
            for cp in cps:
                cp.start()

        def wait(cps):
            for cp in cps:
                cp.wait()

        def process(t, p):
            pv = jnp.full((16,), p, jnp.int32)
            for j in range(2):
                real = jnp.full((16,), 1, jnp.int32) * (t * 2 + j) \
                    < S_REAL_ROWS
                for g in range(8):
                    dv = dstv[p, j, pl.ds(g * 16, 16)]
                    sidx[p, j, pl.ds(g * 16, 16)] = jnp.where(
                        real, dv, U_GARBAGE)
            for j in range(2):
                for g8 in range(8):
                    k0 = jnp.full((16,), j * 128 + g8 * 16, jnp.int32) + iota
                    f2v = plsc.load_gather(S, [pv, k0, col16])
                    f1v = plsc.load_gather(Fb, [pv, k0, col0])
                    sv = f1v + f2v
                    ev = jnp.exp(-jnp.maximum(sv, 0.2 * sv))
                    ebuf[p, j, pl.ds(g8 * 16, 16)] = ev
            for j in range(2):
                cj = jnp.full((16,), j, jnp.int32)

                @pl.loop(0, 32)
                def _(q, _j=j, _cj=cj, _p=p, _pv=pv):
                    i0 = q * 4
                    for u in range(4):
                        evv = plsc.load_gather(
                            ebuf,
                            [_pv, _cj, jnp.full((16,), i0 + u, jnp.int32)])
                        r = _j * 128 + i0 + u
                        S[_p, r, pl.ds(0, 16)] = S[_p, r, pl.ds(0, 16)] * evv
                        S[_p, r, pl.ds(16, 16)] = evv
            for j in range(2):
                pltpu.sync_copy(S.at[p, pl.ds(j * 128, 128)],
                                acc.at[sidx.at[p, j]], add=True)

        @pl.loop(0, 256)
        def _(i):
            S[0, i, pl.ds(0, 16)] = zero16
            S[0, i, pl.ds(16, 16)] = zero16

        zb = s * 2504
        for kk in range(9):
            pltpu.sync_copy(S.at[0, pl.ds(0, 256)],
                            acc.at[pl.ds(zb + kk * 256, 256)])
        pltpu.sync_copy(S.at[0, pl.ds(0, 200)], acc.at[pl.ds(zb + 2304, 200)])
        plsc.subcore_barrier()

        fire(idx_cps(s, 0))
        wait(idx_cps(s, 0))
        fire(gat_cps(0))
        fire(idx_cps(s + 16, 1))

        @pl.loop(0, S_PAIRS)
        def _pair(kk):
            a = s + kk * 32
            wait(idx_cps(a + 16, 1))
            fire(gat_cps(1))
            wait(gat_cps(0))
            process(a, 0)

            @pl.when(kk < S_PAIRS - 1)
            def _():
                fire(idx_cps(a + 32, 0))

            wait(gat_cps(1))

            @pl.when(kk < S_PAIRS - 1)
            def _():
                wait(idx_cps(a + 32, 0))
                fire(gat_cps(0))

            process(a + 16, 1)

            @pl.when(kk < S_PAIRS - 1)
            def _():
                fire(idx_cps(a + 48, 1))

        plsc.subcore_barrier()
        wb = s * 2504

        @pl.when(s < 15)
        def _():
            pltpu.sync_copy(acc.at[pl.ds(wb, 2504)],
                            G_h.at[c, pl.ds(wb, 2504)])

        @pl.when(s == 15)
        def _():
            pltpu.sync_copy(acc.at[pl.ds(15 * 2504, 2440)],
                            G_h.at[c, pl.ds(15 * 2504, 2440)])

    return k(T3, F1, dst3, src3)


def _mean_body(e0, e1, e2, e3, o):
    o[:, 0:HD] = (e0[0] + e1[0] + e2[0] + e3[0]) * 0.25
    o[:, HD:D] = (e0[1] + e1[1] + e2[1] + e3[1]) * 0.25


def _mean(e0, e1, e2, e3):
    blk = 4000
    bs = pl.BlockSpec((2, blk, HD), lambda i: (0, i, 0))
    return pl.pallas_call(
        _mean_body,
        grid=(N_TOTAL // blk,),
        in_specs=[bs] * 4,
        out_specs=pl.BlockSpec((blk, D), lambda i: (i, 0)),
        out_shape=jax.ShapeDtypeStruct((N_TOTAL, D), jnp.float32),
    )(e0, e1, e2, e3)


_UBLK = 2000
_UGRID = NUM_USERS // _UBLK


def _pre_body(x, W, ad, asrc, T, F1):
    h = lax.dot_general(x[...], W[...], (((1,), (0,)), ((), ())),
                        precision=lax.Precision.HIGHEST,
                        preferred_element_type=jnp.float32)
    f1 = jnp.sum(h * ad[...], axis=1, keepdims=True)
    f2 = jnp.sum(h * asrc[...], axis=1, keepdims=True)
    f2b = jnp.broadcast_to(f2, (_UBLK, 16))
    T[0, :, 0:16] = h[:, 0:16]
    T[0, :, 16:32] = f2b
    T[1, :, 0:16] = h[:, 16:32]
    T[1, :, 16:32] = f2b
    F1[...] = jnp.broadcast_to(f1, (_UBLK, 16))


def _pre(x, W, a):
    ad = a[:D, 0].reshape(1, D)
    asrc = a[D:, 0].reshape(1, D)
    full = pl.BlockSpec((1, D), lambda i: (0, 0))
    wfull = pl.BlockSpec((D, D), lambda i: (0, 0))
    return pl.pallas_call(
        _pre_body,
        grid=(_UGRID,),
        in_specs=[pl.BlockSpec((_UBLK, D), lambda i: (i, 0)),
                  wfull, full, full],
        out_specs=(pl.BlockSpec((2, _UBLK, D), lambda i: (0, i, 0)),
                   pl.BlockSpec((_UBLK, 16), lambda i: (i, 0))),
        out_shape=(jax.ShapeDtypeStruct((2, NUM_USERS, D), jnp.float32),
                   jax.ShapeDtypeStruct((NUM_USERS, 16), jnp.float32)),
    )(x, W, ad, asrc)


def _gat_norm(G):
    rs = G[0, :, 16:17] + 1e-9
    g = jnp.concatenate([G[0, :, 0:16], G[1, :, 0:16]], axis=1) / rs
    return jnp.where(g > 0, g, jnp.exp(g) - 1.0)


def _post_body(G, o):
    o[...] = _gat_norm(G)


def _post(G):
    return pl.pallas_call(
        _post_body,
        grid=(_UGRID,),
        in_specs=[pl.BlockSpec((2, _UBLK, D), lambda i: (0, i, 0))],
        out_specs=pl.BlockSpec((_UBLK, D), lambda i: (i, 0)),
        out_shape=jax.ShapeDtypeStruct((NUM_USERS, D), jnp.float32),
    )(G)


def _final_body(G, u, o):
    o[...] = (u[...] + _gat_norm(G)) * 0.5


def _final(G, u):
    return pl.pallas_call(
        _final_body,
        grid=(_UGRID,),
        in_specs=[pl.BlockSpec((2, _UBLK, D), lambda i: (0, i, 0)),
                  pl.BlockSpec((_UBLK, D), lambda i: (i, 0))],
        out_specs=pl.BlockSpec((_UBLK, D), lambda i: (i, 0)),
        out_shape=jax.ShapeDtypeStruct((NUM_USERS, D), jnp.float32),
    )(G, u)


def _pad3(x2d, rows, fill, inner):
    pad = rows - x2d.shape[0]
    full = jnp.concatenate(
        [x2d, jnp.full((pad, 128), fill, x2d.dtype)], axis=0)
    return full.reshape(rows // inner, inner, 128)


def kernel(graph_indices, graph_values, social_indices, user_emb, item_emb,
           W1, a1, W2, a2):
    dst3 = _pad3(graph_indices[0].reshape(-1, 128), G_ROWS, N_TOTAL, 4)
    src3 = _pad3(graph_indices[1].reshape(-1, 128), G_ROWS, 0, 4)
    val3 = _pad3(graph_values.reshape(-1, 128), G_ROWS, 0.0, 4)
    sdst3 = _pad3(social_indices[0].reshape(-1, 128), S_ROWS, 0, 2)
    ssrc3 = _pad3(social_indices[1].reshape(-1, 128), S_ROWS, 0, 2)

    ego0 = jnp.stack([
        jnp.concatenate([user_emb[:, :HD], item_emb[:, :HD]], axis=0),
        jnp.concatenate([user_emb[:, HD:], item_emb[:, HD:]], axis=0)])
    e1 = _spmm(ego0, dst3, src3, val3)
    e2 = _spmm(e1, dst3, src3, val3)
    e3 = _spmm(e2, dst3, src3, val3)
    mean = _mean(ego0, e1, e2, e3)
    user_all = mean[:NUM_USERS]
    item_all = mean[NUM_USERS:]

    T1, F11 = _pre(user_all, W1, a1)
    G1 = _gat_edges(T1, F11, sdst3, ssrc3)
    h = _post(G1)
    T2, F12 = _pre(h, W2, a2)
    G2 = _gat_edges(T2, F12, sdst3, ssrc3)
    out_user = _final(G2, user_all)
    return (out_user, item_all)
